# baseline (speedup 1.0000x reference)
import functools
import math

import jax
import jax.numpy as jnp
from jax import lax
from jax.experimental import pallas as pl
from jax.experimental.pallas import tpu as pltpu
from jax.experimental.pallas import tpu_sc as plsc

_B = 4096
_L = 64
_K = 256
_LANES = 128
_NTILES = 32

_SC_ROWS = 512
_RPT = _SC_ROWS // _NTILES
_TC_ROWS = _B - _SC_ROWS

_C = -0.5 * math.log(2.0 * math.pi)
_NEG = -3.0e38


def _sc_body(zt_hbm, mt_hbm, lvt_hbm, lw_hbm, mx_hbm, s_hbm,
             z_v, m_t, a_t, p_t, lw_v, mx_v, s_v):
    wid = lax.axis_index("s") * 2 + lax.axis_index("c")
    pltpu.sync_copy(zt_hbm.at[wid], z_v)
    pltpu.sync_copy(mt_hbm, m_t)
    pltpu.sync_copy(lvt_hbm, p_t)
    pltpu.sync_copy(lw_hbm, lw_v)

    nkc = _K // 16
    nj = _RPT // 16

    def prep(l, carry):
        for kc in range(nkc):
            sl = pl.ds(16 * kc, 16)
            lw = lw_v[sl]
            lv = p_t[l, sl]
            a_t[l, sl] = (_C + lw) - 0.5 * lv
            p_t[l, sl] = 0.5 * jnp.exp(-lv)
        return carry

    lax.fori_loop(0, _L, prep, 0)

    def lbody(l, carry):
        zv = tuple(z_v[l, pl.ds(16 * j, 16)] for j in range(nj))

        def p1(kc, mxs):
            sl = pl.ds(16 * kc, 16)
            mv = m_t[l, sl]
            av = a_t[l, sl]
            pv = p_t[l, sl]
            mxs = list(mxs)
            for ic in range(4):
                for j in range(nj):
                    ts = []
                    for i in range(4 * ic, 4 * ic + 4):
                        m, a, p = mv[i], av[i], pv[i]
                        d = zv[j] - m
                        ts.append(a - p * d * d)
                    t01 = jnp.maximum(ts[0], ts[1])
                    t23 = jnp.maximum(ts[2], ts[3])
                    mxs[j] = jnp.maximum(mxs[j], jnp.maximum(t01, t23))
            return tuple(mxs)

        mxs = lax.fori_loop(
            0, nkc, p1,
            tuple(jnp.full((16,), _NEG, jnp.float32) for _ in range(nj)),
            unroll=2)

        def p2(kc, ss):
            sl = pl.ds(16 * kc, 16)
            mv = m_t[l, sl]
            av = a_t[l, sl]
            pv = p_t[l, sl]
            ss = list(ss)
            for ic in range(4):
                for j in range(nj):
                    es = []
                    for i in range(4 * ic, 4 * ic + 4):
                        m, a, p = mv[i], av[i], pv[i]
                        d = zv[j] - m
                        es.append(jnp.exp((a - p * d * d) - mxs[j]))
                    e01 = es[0] + es[1]
                    e23 = es[2] + es[3]
                    ss[j] = ss[j] + (e01 + e23)
            return tuple(ss)

        ss = lax.fori_loop(
            0, nkc, p2, tuple(jnp.zeros((16,), jnp.float32) for _ in range(nj)),
            unroll=2)

        for j in range(nj):
            mx_v[l, pl.ds(16 * j, 16)] = mxs[j]
            s_v[l, pl.ds(16 * j, 16)] = ss[j]
        return carry

    lax.fori_loop(0, _L, lbody, 0)
    pltpu.sync_copy(mx_v, mx_hbm.at[wid])
    pltpu.sync_copy(s_v, s_hbm.at[wid])


_sc_mog = functools.partial(
    pl.kernel,
    mesh=plsc.VectorSubcoreMesh(core_axis_name="c", subcore_axis_name="s"),
    out_type=[
        jax.ShapeDtypeStruct((_NTILES, _L, _RPT), jnp.float32),
        jax.ShapeDtypeStruct((_NTILES, _L, _RPT), jnp.float32),
    ],
    scratch_types=[
        pltpu.VMEM((_L, _RPT), jnp.float32),
        pltpu.VMEM((_L, _K), jnp.float32),
        pltpu.VMEM((_L, _K), jnp.float32),
        pltpu.VMEM((_L, _K), jnp.float32),
        pltpu.VMEM((_K,), jnp.float32),
        pltpu.VMEM((_L, _RPT), jnp.float32),
        pltpu.VMEM((_L, _RPT), jnp.float32),
    ],
)(_sc_body)


_NB = 3584
_TC_GRID = _TC_ROWS // _NB


def _tc_body(zt_ref, mt_ref, lvt_ref, lw_ref, o_ref, A_ref, B_ref, C_ref):
    mt = mt_ref[...]
    lvt = lvt_ref[...]
    lw = lw_ref[...]
    wmax = jnp.max(lw)
    lse_w = wmax + jnp.log(jnp.sum(jnp.exp(lw - wmax)))
    p = 0.5 * jnp.exp(-lvt)
    a = (_C + lw) - 0.5 * lvt
    A_ref[...] = a - p * mt * mt
    B_ref[...] = (2.0 * p) * mt
    C_ref[...] = -p

    def lstep(l, carry):
        zrow = zt_ref[pl.ds(l, 1), :]
        zsq = zrow * zrow
        ones = jnp.ones_like(zrow)
        zf = jnp.concatenate([ones, zrow, zsq], axis=0)
        wl = jnp.concatenate([A_ref[pl.ds(l, 1), :],
                              B_ref[pl.ds(l, 1), :],
                              C_ref[pl.ds(l, 1), :]], axis=0)
        t = lax.dot_general(wl, zf, (((0,), (0,)), ((), ())),
                            preferred_element_type=jnp.float32)
        mx = jnp.max(t, axis=0, keepdims=True)
        s = jnp.sum(jnp.exp(t - mx), axis=0, keepdims=True)
        o_ref[pl.ds(l, 1), :] = (mx - lse_w) + jnp.log(s)
        return carry

    lax.fori_loop(0, _L, lstep, 0)


def _tc_main(zt, mt, lvt, lwr):
    return pl.pallas_call(
        _tc_body,
        grid=(_TC_GRID,),
        in_specs=[
            pl.BlockSpec((_L, _NB), lambda i: (0, i)),
            pl.BlockSpec((_L, _K), lambda i: (0, 0)),
            pl.BlockSpec((_L, _K), lambda i: (0, 0)),
            pl.BlockSpec((1, _K), lambda i: (0, 0)),
        ],
        out_specs=pl.BlockSpec((_L, _NB), lambda i: (0, i)),
        out_shape=jax.ShapeDtypeStruct((_L, _TC_ROWS), jnp.float32),
        scratch_shapes=[
            pltpu.VMEM((_L, _K), jnp.float32),
            pltpu.VMEM((_L, _K), jnp.float32),
            pltpu.VMEM((_L, _K), jnp.float32),
        ],
    )(zt, mt, lvt, lwr)


def _fin_body(mx_ref, s_ref, w_ref, o_ref):
    lw = w_ref[...]
    wmax = jnp.max(lw)
    lse_w = wmax + jnp.log(jnp.sum(jnp.exp(lw - wmax)))
    o_ref[...] = (mx_ref[...] - lse_w) + jnp.log(s_ref[...])


def _finish(mx2, s2, wr):
    return pl.pallas_call(
        _fin_body,
        out_shape=jax.ShapeDtypeStruct(mx2.shape, jnp.float32),
    )(mx2, s2, wr)


@jax.jit
def kernel(z, means, logvars, w):
    ws = w.reshape(_K)

    z_sc = z[:_SC_ROWS]
    zt3 = z_sc.reshape(_NTILES, _RPT, _L).transpose(0, 2, 1)
    mx3, s3 = _sc_mog(zt3, means.T, logvars.T, ws)

    zt_tc = z[_SC_ROWS:].T
    out_tc = _tc_main(zt_tc, means.T, logvars.T, ws.reshape(1, _K)).T

    out_sc = (_finish(mx3.reshape(-1, _LANES), s3.reshape(-1, _LANES),
                      ws.reshape(1, _K))
              .reshape(_NTILES, _L, _RPT)
              .transpose(0, 2, 1)
              .reshape(_SC_ROWS, _L))
    return jnp.concatenate([out_sc, out_tc], axis=0)


# speedup vs baseline: 1.2512x; 1.2512x over previous
import functools
import math

import jax
import jax.numpy as jnp
from jax import lax
from jax.experimental import pallas as pl
from jax.experimental.pallas import tpu as pltpu
from jax.experimental.pallas import tpu_sc as plsc

_B = 4096
_L = 64
_K = 256
_LANES = 128
_NTILES = 32

_SC_ROWS = 512
_RPT = _SC_ROWS // _NTILES
_TC_ROWS = _B - _SC_ROWS

_C = -0.5 * math.log(2.0 * math.pi)
_NEG = -3.0e38


def _sc_body(zt_hbm, mt_hbm, lvt_hbm, lw_hbm, mx_hbm, s_hbm,
             z_v, m_t, a_t, p_t, lw_v, mx_v, s_v):
    wid = lax.axis_index("s") * 2 + lax.axis_index("c")
    pltpu.sync_copy(zt_hbm.at[wid], z_v)
    pltpu.sync_copy(mt_hbm, m_t)
    pltpu.sync_copy(lvt_hbm, p_t)
    pltpu.sync_copy(lw_hbm, lw_v)

    nkc = _K // 16
    nj = _RPT // 16

    def prep(l, carry):
        for kc in range(nkc):
            sl = pl.ds(16 * kc, 16)
            lw = lw_v[sl]
            lv = p_t[l, sl]
            a_t[l, sl] = (_C + lw) - 0.5 * lv
            p_t[l, sl] = 0.5 * jnp.exp(-lv)
        return carry

    lax.fori_loop(0, _L, prep, 0)

    def lbody(l, carry):
        zv = tuple(z_v[l, pl.ds(16 * j, 16)] for j in range(nj))

        def p1(kc, mxs):
            sl = pl.ds(16 * kc, 16)
            mv = m_t[l, sl]
            av = a_t[l, sl]
            pv = p_t[l, sl]
            mxs = list(mxs)
            for ic in range(4):
                for j in range(nj):
                    ts = []
                    for i in range(4 * ic, 4 * ic + 4):
                        m, a, p = mv[i], av[i], pv[i]
                        d = zv[j] - m
                        ts.append(a - p * d * d)
                    t01 = jnp.maximum(ts[0], ts[1])
                    t23 = jnp.maximum(ts[2], ts[3])
                    mxs[j] = jnp.maximum(mxs[j], jnp.maximum(t01, t23))
            return tuple(mxs)

        mxs = lax.fori_loop(
            0, nkc, p1,
            tuple(jnp.full((16,), _NEG, jnp.float32) for _ in range(nj)))

        def p2(kc, ss):
            sl = pl.ds(16 * kc, 16)
            mv = m_t[l, sl]
            av = a_t[l, sl]
            pv = p_t[l, sl]
            ss = list(ss)
            for ic in range(4):
                for j in range(nj):
                    es = []
                    for i in range(4 * ic, 4 * ic + 4):
                        m, a, p = mv[i], av[i], pv[i]
                        d = zv[j] - m
                        es.append(jnp.exp((a - p * d * d) - mxs[j]))
                    e01 = es[0] + es[1]
                    e23 = es[2] + es[3]
                    ss[j] = ss[j] + (e01 + e23)
            return tuple(ss)

        ss = lax.fori_loop(
            0, nkc, p2, tuple(jnp.zeros((16,), jnp.float32) for _ in range(nj)))

        for j in range(nj):
            mx_v[l, pl.ds(16 * j, 16)] = mxs[j]
            s_v[l, pl.ds(16 * j, 16)] = ss[j]
        return carry

    lax.fori_loop(0, _L, lbody, 0)
    pltpu.sync_copy(mx_v, mx_hbm.at[wid])
    pltpu.sync_copy(s_v, s_hbm.at[wid])


_sc_mog = functools.partial(
    pl.kernel,
    mesh=plsc.VectorSubcoreMesh(core_axis_name="c", subcore_axis_name="s"),
    out_type=[
        jax.ShapeDtypeStruct((_NTILES, _L, _RPT), jnp.float32),
        jax.ShapeDtypeStruct((_NTILES, _L, _RPT), jnp.float32),
    ],
    scratch_types=[
        pltpu.VMEM((_L, _RPT), jnp.float32),
        pltpu.VMEM((_L, _K), jnp.float32),
        pltpu.VMEM((_L, _K), jnp.float32),
        pltpu.VMEM((_L, _K), jnp.float32),
        pltpu.VMEM((_K,), jnp.float32),
        pltpu.VMEM((_L, _RPT), jnp.float32),
        pltpu.VMEM((_L, _RPT), jnp.float32),
    ],
)(_sc_body)


_NB = 3584
_TC_GRID = _TC_ROWS // _NB


def _tc_body(zt_ref, mt_ref, lvt_ref, lw_ref, o_ref, A_ref, B_ref, C_ref):
    mt = mt_ref[...]
    lvt = lvt_ref[...]
    lw = lw_ref[...]
    wmax = jnp.max(lw)
    lse_w = wmax + jnp.log(jnp.sum(jnp.exp(lw - wmax)))
    p = 0.5 * jnp.exp(-lvt)
    a = (_C + lw) - 0.5 * lvt
    A_ref[...] = a - p * mt * mt
    B_ref[...] = (2.0 * p) * mt
    C_ref[...] = -p

    def lstep(l, carry):
        zrow = zt_ref[pl.ds(l, 1), :]
        zsq = zrow * zrow
        ones = jnp.ones_like(zrow)
        zf = jnp.concatenate([ones, zrow, zsq], axis=0)
        wl = jnp.concatenate([A_ref[pl.ds(l, 1), :],
                              B_ref[pl.ds(l, 1), :],
                              C_ref[pl.ds(l, 1), :]], axis=0)
        t = lax.dot_general(wl, zf, (((0,), (0,)), ((), ())),
                            preferred_element_type=jnp.float32)
        mx = jnp.max(t, axis=0, keepdims=True)
        s = jnp.sum(jnp.exp(t - mx), axis=0, keepdims=True)
        o_ref[pl.ds(l, 1), :] = (mx - lse_w) + jnp.log(s)
        return carry

    lax.fori_loop(0, _L, lstep, 0)


def _tc_main(zt, mt, lvt, lwr):
    return pl.pallas_call(
        _tc_body,
        grid=(_TC_GRID,),
        in_specs=[
            pl.BlockSpec((_L, _NB), lambda i: (0, i)),
            pl.BlockSpec((_L, _K), lambda i: (0, 0)),
            pl.BlockSpec((_L, _K), lambda i: (0, 0)),
            pl.BlockSpec((1, _K), lambda i: (0, 0)),
        ],
        out_specs=pl.BlockSpec((_L, _NB), lambda i: (0, i)),
        out_shape=jax.ShapeDtypeStruct((_L, _TC_ROWS), jnp.float32),
        scratch_shapes=[
            pltpu.VMEM((_L, _K), jnp.float32),
            pltpu.VMEM((_L, _K), jnp.float32),
            pltpu.VMEM((_L, _K), jnp.float32),
        ],
    )(zt, mt, lvt, lwr)


def _fin_body(mx_ref, s_ref, w_ref, o_ref):
    lw = w_ref[...]
    wmax = jnp.max(lw)
    lse_w = wmax + jnp.log(jnp.sum(jnp.exp(lw - wmax)))
    o_ref[...] = (mx_ref[...] - lse_w) + jnp.log(s_ref[...])


def _finish(mx2, s2, wr):
    return pl.pallas_call(
        _fin_body,
        out_shape=jax.ShapeDtypeStruct(mx2.shape, jnp.float32),
    )(mx2, s2, wr)


@jax.jit
def kernel(z, means, logvars, w):
    ws = w.reshape(_K)

    z_sc = z[:_SC_ROWS]
    zt3 = z_sc.reshape(_NTILES, _RPT, _L).transpose(0, 2, 1)
    mx3, s3 = _sc_mog(zt3, means.T, logvars.T, ws)

    zt_tc = z[_SC_ROWS:].T
    out_tc = _tc_main(zt_tc, means.T, logvars.T, ws.reshape(1, _K)).T

    out_sc = (_finish(mx3.reshape(-1, _LANES), s3.reshape(-1, _LANES),
                      ws.reshape(1, _K))
              .reshape(_NTILES, _L, _RPT)
              .transpose(0, 2, 1)
              .reshape(_SC_ROWS, _L))
    return jnp.concatenate([out_sc, out_tc], axis=0)


# speedup vs baseline: 1.3961x; 1.1158x over previous
import functools
import math

import jax
import jax.numpy as jnp
from jax import lax
from jax.experimental import pallas as pl
from jax.experimental.pallas import tpu as pltpu
from jax.experimental.pallas import tpu_sc as plsc

_B = 4096
_L = 64
_K = 256
_LANES = 128
_NTILES = 32

_SC_ROWS = 256
_BPT = 16
_NBG = _SC_ROWS // _BPT
_LH = _L // 2
_TC_ROWS = _B - _SC_ROWS

_C = -0.5 * math.log(2.0 * math.pi)
_NEG = -3.0e38


def _sc_body(zt_hbm, mt_hbm, lvt_hbm, lw_hbm, mx_hbm, s_hbm,
             z_v, m_t, a_t, p_t, lw_v, mx_v, s_v):
    bg = lax.axis_index("s")
    lh = lax.axis_index("c")
    wid = bg * 2 + lh
    l0 = lh * _LH
    pltpu.sync_copy(zt_hbm.at[bg], z_v)
    pltpu.sync_copy(mt_hbm, m_t)
    pltpu.sync_copy(lvt_hbm, p_t)
    pltpu.sync_copy(lw_hbm, lw_v)

    nkc = _K // 16

    def prep(ll, carry):
        l = l0 + ll
        for kc in range(nkc):
            sl = pl.ds(16 * kc, 16)
            lw = lw_v[sl]
            lv = p_t[l, sl]
            a_t[l, sl] = (_C + lw) - 0.5 * lv
            p_t[l, sl] = 0.5 * jnp.exp(-lv)
        return carry

    lax.fori_loop(0, _LH, prep, 0)

    def lbody(ll, carry):
        l = l0 + ll
        zv = z_v[l, :]

        def p1(kc, mx):
            sl = pl.ds(16 * kc, 16)
            mv = m_t[l, sl]
            av = a_t[l, sl]
            pv = p_t[l, sl]
            for ic in range(4):
                ts = []
                for i in range(4 * ic, 4 * ic + 4):
                    m, a, p = mv[i], av[i], pv[i]
                    d = zv - m
                    ts.append(a - p * d * d)
                t01 = jnp.maximum(ts[0], ts[1])
                t23 = jnp.maximum(ts[2], ts[3])
                mx = jnp.maximum(mx, jnp.maximum(t01, t23))
            return mx

        mx = lax.fori_loop(0, nkc, p1, jnp.full((16,), _NEG, jnp.float32))

        def p2(kc, s):
            sl = pl.ds(16 * kc, 16)
            mv = m_t[l, sl]
            av = a_t[l, sl]
            pv = p_t[l, sl]
            for ic in range(4):
                es = []
                for i in range(4 * ic, 4 * ic + 4):
                    m, a, p = mv[i], av[i], pv[i]
                    d = zv - m
                    es.append(jnp.exp((a - p * d * d) - mx))
                s = s + ((es[0] + es[1]) + (es[2] + es[3]))
            return s

        s = lax.fori_loop(0, nkc, p2, jnp.zeros((16,), jnp.float32))

        mx_v[ll, :] = mx
        s_v[ll, :] = s
        return carry

    lax.fori_loop(0, _LH, lbody, 0)
    pltpu.sync_copy(mx_v, mx_hbm.at[wid])
    pltpu.sync_copy(s_v, s_hbm.at[wid])


_sc_mog = functools.partial(
    pl.kernel,
    mesh=plsc.VectorSubcoreMesh(core_axis_name="c", subcore_axis_name="s"),
    out_type=[
        jax.ShapeDtypeStruct((_NTILES, _LH, _BPT), jnp.float32),
        jax.ShapeDtypeStruct((_NTILES, _LH, _BPT), jnp.float32),
    ],
    scratch_types=[
        pltpu.VMEM((_L, _BPT), jnp.float32),
        pltpu.VMEM((_L, _K), jnp.float32),
        pltpu.VMEM((_L, _K), jnp.float32),
        pltpu.VMEM((_L, _K), jnp.float32),
        pltpu.VMEM((_K,), jnp.float32),
        pltpu.VMEM((_LH, _BPT), jnp.float32),
        pltpu.VMEM((_LH, _BPT), jnp.float32),
    ],
)(_sc_body)


_NB = _TC_ROWS
_TC_GRID = _TC_ROWS // _NB


def _tc_body(zt_ref, mt_ref, lvt_ref, lw_ref, o_ref, A_ref, B_ref, C_ref):
    mt = mt_ref[...]
    lvt = lvt_ref[...]
    lw = lw_ref[...]
    wmax = jnp.max(lw)
    lse_w = wmax + jnp.log(jnp.sum(jnp.exp(lw - wmax)))
    p = 0.5 * jnp.exp(-lvt)
    a = (_C + lw) - 0.5 * lvt
    A_ref[...] = a - p * mt * mt
    B_ref[...] = (2.0 * p) * mt
    C_ref[...] = -p

    def lstep(l, carry):
        zrow = zt_ref[pl.ds(l, 1), :]
        zsq = zrow * zrow
        ones = jnp.ones_like(zrow)
        zf = jnp.concatenate([ones, zrow, zsq], axis=0)
        wl = jnp.concatenate([A_ref[pl.ds(l, 1), :],
                              B_ref[pl.ds(l, 1), :],
                              C_ref[pl.ds(l, 1), :]], axis=0)
        t = lax.dot_general(wl, zf, (((0,), (0,)), ((), ())),
                            preferred_element_type=jnp.float32)
        mx = jnp.max(t, axis=0, keepdims=True)
        s = jnp.sum(jnp.exp(t - mx), axis=0, keepdims=True)
        o_ref[pl.ds(l, 1), :] = (mx - lse_w) + jnp.log(s)
        return carry

    lax.fori_loop(0, _L, lstep, 0)


def _tc_main(zt, mt, lvt, lwr):
    return pl.pallas_call(
        _tc_body,
        grid=(_TC_GRID,),
        in_specs=[
            pl.BlockSpec((_L, _NB), lambda i: (0, i)),
            pl.BlockSpec((_L, _K), lambda i: (0, 0)),
            pl.BlockSpec((_L, _K), lambda i: (0, 0)),
            pl.BlockSpec((1, _K), lambda i: (0, 0)),
        ],
        out_specs=pl.BlockSpec((_L, _NB), lambda i: (0, i)),
        out_shape=jax.ShapeDtypeStruct((_L, _TC_ROWS), jnp.float32),
        scratch_shapes=[
            pltpu.VMEM((_L, _K), jnp.float32),
            pltpu.VMEM((_L, _K), jnp.float32),
            pltpu.VMEM((_L, _K), jnp.float32),
        ],
    )(zt, mt, lvt, lwr)


def _fin_body(mx_ref, s_ref, w_ref, o_ref):
    lw = w_ref[...]
    wmax = jnp.max(lw)
    lse_w = wmax + jnp.log(jnp.sum(jnp.exp(lw - wmax)))
    o_ref[...] = (mx_ref[...] - lse_w) + jnp.log(s_ref[...])


def _finish(mx2, s2, wr):
    return pl.pallas_call(
        _fin_body,
        out_shape=jax.ShapeDtypeStruct(mx2.shape, jnp.float32),
    )(mx2, s2, wr)


@jax.jit
def kernel(z, means, logvars, w):
    ws = w.reshape(_K)

    z_sc = z[:_SC_ROWS]
    zt3 = z_sc.reshape(_NBG, _BPT, _L).transpose(0, 2, 1)
    mx3, s3 = _sc_mog(zt3, means.T, logvars.T, ws)

    zt_tc = z[_SC_ROWS:].T
    out_tc = _tc_main(zt_tc, means.T, logvars.T, ws.reshape(1, _K)).T

    out_sc = (_finish(mx3.reshape(-1, _LANES), s3.reshape(-1, _LANES),
                      ws.reshape(1, _K))
              .reshape(_NBG, 2, _LH, _BPT)
              .transpose(0, 3, 1, 2)
              .reshape(_SC_ROWS, _L))
    return jnp.concatenate([out_sc, out_tc], axis=0)


# speedup vs baseline: 1.4263x; 1.0216x over previous
import functools
import math

import jax
import jax.numpy as jnp
from jax import lax
from jax.experimental import pallas as pl
from jax.experimental.pallas import tpu as pltpu
from jax.experimental.pallas import tpu_sc as plsc

_B = 4096
_L = 64
_K = 256
_LANES = 128
_NTILES = 32

_SC_ROWS = 256
_BPT = 16
_NBG = _SC_ROWS // _BPT
_LH = _L // 2
_TC_ROWS = _B - _SC_ROWS

_C = -0.5 * math.log(2.0 * math.pi)
_NEG = -3.0e38


def _sc_body(zt_hbm, mt_hbm, lvt_hbm, lw_hbm, mx_hbm, s_hbm,
             z_v, m_t, a_t, p_t, lw_v, mx_v, s_v):
    bg = lax.axis_index("s")
    lh = lax.axis_index("c")
    wid = bg * 2 + lh
    l0 = lh * _LH
    pltpu.sync_copy(zt_hbm.at[bg], z_v)
    pltpu.sync_copy(mt_hbm, m_t)
    pltpu.sync_copy(lvt_hbm, p_t)
    pltpu.sync_copy(lw_hbm, lw_v)

    nkc = _K // 16

    def prep(ll, carry):
        l = l0 + ll
        for kc in range(nkc):
            sl = pl.ds(16 * kc, 16)
            lw = lw_v[sl]
            lv = p_t[l, sl]
            a_t[l, sl] = (_C + lw) - 0.5 * lv
            p_t[l, sl] = 0.5 * jnp.exp(-lv)
        return carry

    lax.fori_loop(0, _LH, prep, 0)

    def lbody(ll, carry):
        l = l0 + ll
        zv = z_v[l, :]

        def p1(kc, mx):
            sl = pl.ds(16 * kc, 16)
            mv = m_t[l, sl]
            av = a_t[l, sl]
            pv = p_t[l, sl]
            for ic in range(4):
                ts = []
                for i in range(4 * ic, 4 * ic + 4):
                    m, a, p = mv[i], av[i], pv[i]
                    d = zv - m
                    ts.append(a - p * d * d)
                t01 = jnp.maximum(ts[0], ts[1])
                t23 = jnp.maximum(ts[2], ts[3])
                mx = jnp.maximum(mx, jnp.maximum(t01, t23))
            return mx

        mx = lax.fori_loop(0, nkc, p1, jnp.full((16,), _NEG, jnp.float32))

        def p2(kc, s):
            sl = pl.ds(16 * kc, 16)
            mv = m_t[l, sl]
            av = a_t[l, sl]
            pv = p_t[l, sl]
            for ic in range(4):
                es = []
                for i in range(4 * ic, 4 * ic + 4):
                    m, a, p = mv[i], av[i], pv[i]
                    d = zv - m
                    es.append(jnp.exp((a - p * d * d) - mx))
                s = s + ((es[0] + es[1]) + (es[2] + es[3]))
            return s

        s = lax.fori_loop(0, nkc, p2, jnp.zeros((16,), jnp.float32))

        mx_v[ll, :] = mx
        s_v[ll, :] = s
        return carry

    lax.fori_loop(0, _LH, lbody, 0)
    pltpu.sync_copy(mx_v, mx_hbm.at[wid])
    pltpu.sync_copy(s_v, s_hbm.at[wid])


_sc_mog = functools.partial(
    pl.kernel,
    mesh=plsc.VectorSubcoreMesh(core_axis_name="c", subcore_axis_name="s"),
    out_type=[
        jax.ShapeDtypeStruct((_NTILES, _LH, _BPT), jnp.float32),
        jax.ShapeDtypeStruct((_NTILES, _LH, _BPT), jnp.float32),
    ],
    scratch_types=[
        pltpu.VMEM((_L, _BPT), jnp.float32),
        pltpu.VMEM((_L, _K), jnp.float32),
        pltpu.VMEM((_L, _K), jnp.float32),
        pltpu.VMEM((_L, _K), jnp.float32),
        pltpu.VMEM((_K,), jnp.float32),
        pltpu.VMEM((_LH, _BPT), jnp.float32),
        pltpu.VMEM((_LH, _BPT), jnp.float32),
    ],
)(_sc_body)


_NB = _TC_ROWS
_TC_GRID = _TC_ROWS // _NB


def _tc_body(zt_ref, mt_ref, lvt_ref, lw_ref, o_ref, A_ref, B_ref, C_ref):
    mt = mt_ref[...]
    lvt = lvt_ref[...]
    lw = lw_ref[...]
    wmax = jnp.max(lw)
    lse_w = wmax + jnp.log(jnp.sum(jnp.exp(lw - wmax)))
    p = 0.5 * jnp.exp(-lvt)
    a = (_C + lw) - 0.5 * lvt
    A_ref[...] = a - p * mt * mt
    B_ref[...] = (2.0 * p) * mt
    C_ref[...] = -p

    def lstep(l, carry):
        zrow = zt_ref[pl.ds(l, 1), pl.ds(_SC_ROWS, _NB)]
        zsq = zrow * zrow
        ones = jnp.ones_like(zrow)
        zf = jnp.concatenate([ones, zrow, zsq], axis=0)
        wl = jnp.concatenate([A_ref[pl.ds(l, 1), :],
                              B_ref[pl.ds(l, 1), :],
                              C_ref[pl.ds(l, 1), :]], axis=0)
        t = lax.dot_general(wl, zf, (((0,), (0,)), ((), ())),
                            preferred_element_type=jnp.float32)
        mx = jnp.max(t, axis=0, keepdims=True)
        s = jnp.sum(jnp.exp(t - mx), axis=0, keepdims=True)
        o_ref[pl.ds(l, 1), :] = (mx - lse_w) + jnp.log(s)
        return carry

    lax.fori_loop(0, _L, lstep, 0)


def _tc_main(zt, mt, lvt, lwr):
    return pl.pallas_call(
        _tc_body,
        grid=(_TC_GRID,),
        in_specs=[
            pl.BlockSpec((_L, _B), lambda i: (0, 0)),
            pl.BlockSpec((_L, _K), lambda i: (0, 0)),
            pl.BlockSpec((_L, _K), lambda i: (0, 0)),
            pl.BlockSpec((1, _K), lambda i: (0, 0)),
        ],
        out_specs=pl.BlockSpec((_L, _NB), lambda i: (0, i)),
        out_shape=jax.ShapeDtypeStruct((_L, _TC_ROWS), jnp.float32),
        scratch_shapes=[
            pltpu.VMEM((_L, _K), jnp.float32),
            pltpu.VMEM((_L, _K), jnp.float32),
            pltpu.VMEM((_L, _K), jnp.float32),
        ],
    )(zt, mt, lvt, lwr)


def _fin_body(mx_ref, s_ref, w_ref, o_ref):
    lw = w_ref[...]
    wmax = jnp.max(lw)
    lse_w = wmax + jnp.log(jnp.sum(jnp.exp(lw - wmax)))
    o_ref[...] = (mx_ref[...] - lse_w) + jnp.log(s_ref[...])


def _finish(mx2, s2, wr):
    return pl.pallas_call(
        _fin_body,
        out_shape=jax.ShapeDtypeStruct(mx2.shape, jnp.float32),
    )(mx2, s2, wr)


@jax.jit
def kernel(z, means, logvars, w):
    ws = w.reshape(_K)

    zt = z.T
    mt = means.T
    lvt = logvars.T

    zt3 = z[:_SC_ROWS].reshape(_NBG, _BPT, _L).transpose(0, 2, 1)
    mx3, s3 = _sc_mog(zt3, mt, lvt, ws)

    out_tc = _tc_main(zt, mt, lvt, ws.reshape(1, _K)).T

    out_sc = (_finish(mx3.reshape(-1, _LANES), s3.reshape(-1, _LANES),
                      ws.reshape(1, _K))
              .reshape(_NBG, 2, _LH, _BPT)
              .transpose(0, 3, 1, 2)
              .reshape(_SC_ROWS, _L))
    return jnp.concatenate([out_sc, out_tc], axis=0)


# speedup vs baseline: 1.4302x; 1.0027x over previous
import functools
import math

import jax
import jax.numpy as jnp
from jax import lax
from jax.experimental import pallas as pl
from jax.experimental.pallas import tpu as pltpu
from jax.experimental.pallas import tpu_sc as plsc

_B = 4096
_L = 64
_K = 256
_LANES = 128
_NTILES = 32

_SC_ROWS = 256
_BPT = 16
_NBG = _SC_ROWS // _BPT
_LH = _L // 2
_TC_ROWS = _B - _SC_ROWS

_C = -0.5 * math.log(2.0 * math.pi)
_NEG = -3.0e38


def _sc_body(zt_hbm, mt_hbm, lvt_hbm, lw_hbm, mx_hbm, s_hbm,
             z_v, m_t, a_t, p_t, lw_v, mx_v, s_v):
    bg = lax.axis_index("s")
    lh = lax.axis_index("c")
    wid = bg * 2 + lh
    l0 = lh * _LH
    pltpu.sync_copy(zt_hbm.at[bg], z_v)
    pltpu.sync_copy(mt_hbm, m_t)
    pltpu.sync_copy(lvt_hbm, p_t)
    pltpu.sync_copy(lw_hbm, lw_v)

    nkc = _K // 16

    def prep(ll, carry):
        l = l0 + ll
        for kc in range(nkc):
            sl = pl.ds(16 * kc, 16)
            lw = lw_v[sl]
            lv = p_t[l, sl]
            a_t[l, sl] = (_C + lw) - 0.5 * lv
            p_t[l, sl] = 0.5 * jnp.exp(-lv)
        return carry

    lax.fori_loop(0, _LH, prep, 0)

    def lbody(ll, carry):
        l = l0 + ll
        zv = z_v[l, :]

        def p1(kc, mx):
            sl = pl.ds(16 * kc, 16)
            mv = m_t[l, sl]
            av = a_t[l, sl]
            pv = p_t[l, sl]
            for ic in range(4):
                ts = []
                for i in range(4 * ic, 4 * ic + 4):
                    m, a, p = mv[i], av[i], pv[i]
                    d = zv - m
                    ts.append(a - p * d * d)
                t01 = jnp.maximum(ts[0], ts[1])
                t23 = jnp.maximum(ts[2], ts[3])
                mx = jnp.maximum(mx, jnp.maximum(t01, t23))
            return mx

        mx = lax.fori_loop(0, nkc, p1, jnp.full((16,), _NEG, jnp.float32))

        def p2(kc, s):
            sl = pl.ds(16 * kc, 16)
            mv = m_t[l, sl]
            av = a_t[l, sl]
            pv = p_t[l, sl]
            for ic in range(4):
                es = []
                for i in range(4 * ic, 4 * ic + 4):
                    m, a, p = mv[i], av[i], pv[i]
                    d = zv - m
                    es.append(jnp.exp((a - p * d * d) - mx))
                s = s + ((es[0] + es[1]) + (es[2] + es[3]))
            return s

        s = lax.fori_loop(0, nkc, p2, jnp.zeros((16,), jnp.float32))

        mx_v[ll, :] = mx
        s_v[ll, :] = s
        return carry

    lax.fori_loop(0, _LH, lbody, 0)
    pltpu.sync_copy(mx_v, mx_hbm.at[wid])
    pltpu.sync_copy(s_v, s_hbm.at[wid])


_sc_mog = functools.partial(
    pl.kernel,
    mesh=plsc.VectorSubcoreMesh(core_axis_name="c", subcore_axis_name="s"),
    out_type=[
        jax.ShapeDtypeStruct((_NTILES, _LH, _BPT), jnp.float32),
        jax.ShapeDtypeStruct((_NTILES, _LH, _BPT), jnp.float32),
    ],
    scratch_types=[
        pltpu.VMEM((_L, _BPT), jnp.float32),
        pltpu.VMEM((_L, _K), jnp.float32),
        pltpu.VMEM((_L, _K), jnp.float32),
        pltpu.VMEM((_L, _K), jnp.float32),
        pltpu.VMEM((_K,), jnp.float32),
        pltpu.VMEM((_LH, _BPT), jnp.float32),
        pltpu.VMEM((_LH, _BPT), jnp.float32),
    ],
)(_sc_body)


_NB = _TC_ROWS
_TC_GRID = _TC_ROWS // _NB


def _tc_body(zt_ref, mt_ref, lvt_ref, lw_ref, o_ref, A_ref, B_ref, C_ref):
    mt = mt_ref[...]
    lvt = lvt_ref[...]
    lw = lw_ref[...]
    wmax = jnp.max(lw)
    lse_w = wmax + jnp.log(jnp.sum(jnp.exp(lw - wmax)))
    p = 0.5 * jnp.exp(-lvt)
    a = (_C + lw) - 0.5 * lvt
    A_ref[...] = a - p * mt * mt
    B_ref[...] = (2.0 * p) * mt
    C_ref[...] = -p

    def lstep(l, carry):
        zrow = zt_ref[pl.ds(l, 1), pl.ds(_SC_ROWS, _NB)]
        zsq = zrow * zrow
        ones = jnp.ones_like(zrow)
        zf = jnp.concatenate([ones, zrow, zsq], axis=0)
        wl = jnp.concatenate([A_ref[pl.ds(l, 1), :],
                              B_ref[pl.ds(l, 1), :],
                              C_ref[pl.ds(l, 1), :]], axis=0)
        t = lax.dot_general(wl, zf, (((0,), (0,)), ((), ())),
                            preferred_element_type=jnp.float32)
        mx = jnp.max(t, axis=0, keepdims=True)
        s = jnp.sum(jnp.exp(t - mx), axis=0, keepdims=True)
        o_ref[pl.ds(l, 1), :] = (mx - lse_w) + jnp.log(s)
        return carry

    lax.fori_loop(0, _L, lstep, 0, unroll=2)


def _tc_main(zt, mt, lvt, lwr):
    return pl.pallas_call(
        _tc_body,
        grid=(_TC_GRID,),
        in_specs=[
            pl.BlockSpec((_L, _B), lambda i: (0, 0)),
            pl.BlockSpec((_L, _K), lambda i: (0, 0)),
            pl.BlockSpec((_L, _K), lambda i: (0, 0)),
            pl.BlockSpec((1, _K), lambda i: (0, 0)),
        ],
        out_specs=pl.BlockSpec((_L, _NB), lambda i: (0, i)),
        out_shape=jax.ShapeDtypeStruct((_L, _TC_ROWS), jnp.float32),
        scratch_shapes=[
            pltpu.VMEM((_L, _K), jnp.float32),
            pltpu.VMEM((_L, _K), jnp.float32),
            pltpu.VMEM((_L, _K), jnp.float32),
        ],
    )(zt, mt, lvt, lwr)


def _fin_body(mx_ref, s_ref, w_ref, o_ref):
    lw = w_ref[...]
    wmax = jnp.max(lw)
    lse_w = wmax + jnp.log(jnp.sum(jnp.exp(lw - wmax)))
    o_ref[...] = (mx_ref[...] - lse_w) + jnp.log(s_ref[...])


def _finish(mx2, s2, wr):
    return pl.pallas_call(
        _fin_body,
        out_shape=jax.ShapeDtypeStruct(mx2.shape, jnp.float32),
    )(mx2, s2, wr)


@jax.jit
def kernel(z, means, logvars, w):
    ws = w.reshape(_K)

    zt = z.T
    mt = means.T
    lvt = logvars.T

    zt3 = z[:_SC_ROWS].reshape(_NBG, _BPT, _L).transpose(0, 2, 1)
    mx3, s3 = _sc_mog(zt3, mt, lvt, ws)

    out_tc = _tc_main(zt, mt, lvt, ws.reshape(1, _K)).T

    out_sc = (_finish(mx3.reshape(-1, _LANES), s3.reshape(-1, _LANES),
                      ws.reshape(1, _K))
              .reshape(_NBG, 2, _LH, _BPT)
              .transpose(0, 3, 1, 2)
              .reshape(_SC_ROWS, _L))
    return jnp.concatenate([out_sc, out_tc], axis=0)


# speedup vs baseline: 1.4793x; 1.0343x over previous
import functools
import math

import jax
import jax.numpy as jnp
from jax import lax
from jax.experimental import pallas as pl
from jax.experimental.pallas import tpu as pltpu
from jax.experimental.pallas import tpu_sc as plsc

_B = 4096
_L = 64
_K = 256
_LANES = 128
_NTILES = 32

_SC_ROWS = 512
_BPT = 32
_NBG = _SC_ROWS // _BPT
_LH = _L // 2
_TC_ROWS = _B - _SC_ROWS

_C = -0.5 * math.log(2.0 * math.pi)
_NEG = -3.0e38


def _sc_body(zt_hbm, mt_hbm, lvt_hbm, lw_hbm, mx_hbm, s_hbm,
             z_v, m_t, a_t, p_t, lw_v, mx_v, s_v, t_v):
    bg = lax.axis_index("s")
    lh = lax.axis_index("c")
    wid = bg * 2 + lh
    l0 = lh * _LH
    pltpu.sync_copy(zt_hbm.at[bg], z_v)
    pltpu.sync_copy(mt_hbm, m_t)
    pltpu.sync_copy(lvt_hbm, p_t)
    pltpu.sync_copy(lw_hbm, lw_v)

    nkc = _K // 16

    def prep(ll, carry):
        l = l0 + ll
        for kc in range(nkc):
            sl = pl.ds(16 * kc, 16)
            lw = lw_v[sl]
            lv = p_t[l, sl]
            a_t[l, sl] = (_C + lw) - 0.5 * lv
            p_t[l, sl] = 0.5 * jnp.exp(-lv)
        return carry

    lax.fori_loop(0, _LH, prep, 0)

    nj = _BPT // 16

    def lbody(ll, carry):
        l = l0 + ll
        zv = tuple(z_v[l, pl.ds(16 * j, 16)] for j in range(nj))

        def p1(kc, mxs):
            sl = pl.ds(16 * kc, 16)
            mv = m_t[l, sl]
            av = a_t[l, sl]
            pv = p_t[l, sl]
            mxs = list(mxs)
            for ic in range(4):
                for j in range(nj):
                    ts = []
                    for i in range(4 * ic, 4 * ic + 4):
                        m, a, p = mv[i], av[i], pv[i]
                        d = zv[j] - m
                        t = a - p * d * d
                        t_v[16 * kc + i, pl.ds(16 * j, 16)] = t
                        ts.append(t)
                    t01 = jnp.maximum(ts[0], ts[1])
                    t23 = jnp.maximum(ts[2], ts[3])
                    mxs[j] = jnp.maximum(mxs[j], jnp.maximum(t01, t23))
            return tuple(mxs)

        mxs = lax.fori_loop(
            0, nkc, p1,
            tuple(jnp.full((16,), _NEG, jnp.float32) for _ in range(nj)))

        def p2(kc, ss):
            ss = list(ss)
            for ic in range(4):
                for j in range(nj):
                    es = []
                    for i in range(4 * ic, 4 * ic + 4):
                        t = t_v[16 * kc + i, pl.ds(16 * j, 16)]
                        es.append(jnp.exp(t - mxs[j]))
                    ss[j] = ss[j] + ((es[0] + es[1]) + (es[2] + es[3]))
            return tuple(ss)

        ss = lax.fori_loop(
            0, nkc, p2, tuple(jnp.zeros((16,), jnp.float32) for _ in range(nj)))

        for j in range(nj):
            mx_v[ll, pl.ds(16 * j, 16)] = mxs[j]
            s_v[ll, pl.ds(16 * j, 16)] = ss[j]
        return carry

    lax.fori_loop(0, _LH, lbody, 0)
    pltpu.sync_copy(mx_v, mx_hbm.at[wid])
    pltpu.sync_copy(s_v, s_hbm.at[wid])


_sc_mog = functools.partial(
    pl.kernel,
    mesh=plsc.VectorSubcoreMesh(core_axis_name="c", subcore_axis_name="s"),
    out_type=[
        jax.ShapeDtypeStruct((_NTILES, _LH, _BPT), jnp.float32),
        jax.ShapeDtypeStruct((_NTILES, _LH, _BPT), jnp.float32),
    ],
    scratch_types=[
        pltpu.VMEM((_L, _BPT), jnp.float32),
        pltpu.VMEM((_L, _K), jnp.float32),
        pltpu.VMEM((_L, _K), jnp.float32),
        pltpu.VMEM((_L, _K), jnp.float32),
        pltpu.VMEM((_K,), jnp.float32),
        pltpu.VMEM((_LH, _BPT), jnp.float32),
        pltpu.VMEM((_LH, _BPT), jnp.float32),
        pltpu.VMEM((_K, _BPT), jnp.float32),
    ],
)(_sc_body)


_NB = _TC_ROWS
_TC_GRID = _TC_ROWS // _NB


def _tc_body(zt_ref, mt_ref, lvt_ref, lw_ref, o_ref, A_ref, B_ref, C_ref):
    mt = mt_ref[...]
    lvt = lvt_ref[...]
    lw = lw_ref[...]
    wmax = jnp.max(lw)
    lse_w = wmax + jnp.log(jnp.sum(jnp.exp(lw - wmax)))
    p = 0.5 * jnp.exp(-lvt)
    a = (_C + lw) - 0.5 * lvt
    A_ref[...] = a - p * mt * mt
    B_ref[...] = (2.0 * p) * mt
    C_ref[...] = -p

    def lstep(l, carry):
        zrow = zt_ref[pl.ds(l, 1), pl.ds(_SC_ROWS, _NB)]
        zsq = zrow * zrow
        ones = jnp.ones_like(zrow)
        zf = jnp.concatenate([ones, zrow, zsq], axis=0)
        wl = jnp.concatenate([A_ref[pl.ds(l, 1), :],
                              B_ref[pl.ds(l, 1), :],
                              C_ref[pl.ds(l, 1), :]], axis=0)
        t = lax.dot_general(wl, zf, (((0,), (0,)), ((), ())),
                            preferred_element_type=jnp.float32)
        mx = jnp.max(t, axis=0, keepdims=True)
        s = jnp.sum(jnp.exp(t - mx), axis=0, keepdims=True)
        o_ref[pl.ds(l, 1), :] = (mx - lse_w) + jnp.log(s)
        return carry

    lax.fori_loop(0, _L, lstep, 0, unroll=2)


def _tc_main(zt, mt, lvt, lwr):
    return pl.pallas_call(
        _tc_body,
        grid=(_TC_GRID,),
        in_specs=[
            pl.BlockSpec((_L, _B), lambda i: (0, 0)),
            pl.BlockSpec((_L, _K), lambda i: (0, 0)),
            pl.BlockSpec((_L, _K), lambda i: (0, 0)),
            pl.BlockSpec((1, _K), lambda i: (0, 0)),
        ],
        out_specs=pl.BlockSpec((_L, _NB), lambda i: (0, i)),
        out_shape=jax.ShapeDtypeStruct((_L, _TC_ROWS), jnp.float32),
        scratch_shapes=[
            pltpu.VMEM((_L, _K), jnp.float32),
            pltpu.VMEM((_L, _K), jnp.float32),
            pltpu.VMEM((_L, _K), jnp.float32),
        ],
    )(zt, mt, lvt, lwr)


def _fin_body(mx_ref, s_ref, w_ref, o_ref):
    lw = w_ref[...]
    wmax = jnp.max(lw)
    lse_w = wmax + jnp.log(jnp.sum(jnp.exp(lw - wmax)))
    o_ref[...] = (mx_ref[...] - lse_w) + jnp.log(s_ref[...])


def _finish(mx2, s2, wr):
    return pl.pallas_call(
        _fin_body,
        out_shape=jax.ShapeDtypeStruct(mx2.shape, jnp.float32),
    )(mx2, s2, wr)


@jax.jit
def kernel(z, means, logvars, w):
    ws = w.reshape(_K)

    zt = z.T
    mt = means.T
    lvt = logvars.T

    zt3 = z[:_SC_ROWS].reshape(_NBG, _BPT, _L).transpose(0, 2, 1)
    mx3, s3 = _sc_mog(zt3, mt, lvt, ws)

    out_tc = _tc_main(zt, mt, lvt, ws.reshape(1, _K)).T

    out_sc = (_finish(mx3.reshape(-1, _LANES), s3.reshape(-1, _LANES),
                      ws.reshape(1, _K))
              .reshape(_NBG, 2, _LH, _BPT)
              .transpose(0, 3, 1, 2)
              .reshape(_SC_ROWS, _L))
    return jnp.concatenate([out_sc, out_tc], axis=0)


# speedup vs baseline: 1.4825x; 1.0022x over previous
import functools
import math

import jax
import jax.numpy as jnp
from jax import lax
from jax.experimental import pallas as pl
from jax.experimental.pallas import tpu as pltpu
from jax.experimental.pallas import tpu_sc as plsc

_B = 4096
_L = 64
_K = 256
_LANES = 128
_NTILES = 32

_SC_ROWS = 512
_BPT = 32
_NBG = _SC_ROWS // _BPT
_LH = _L // 2
_TC_ROWS = _B - _SC_ROWS

_C = -0.5 * math.log(2.0 * math.pi)
_NEG = -3.0e38


def _sc_body(zt_hbm, mt_hbm, lvt_hbm, lw_hbm, mx_hbm, s_hbm,
             z_v, m_t, a_t, p_t, lw_v, mx_v, s_v, t_v):
    bg = lax.axis_index("s")
    lh = lax.axis_index("c")
    wid = bg * 2 + lh
    l0 = lh * _LH
    pltpu.sync_copy(zt_hbm.at[bg], z_v)
    pltpu.sync_copy(mt_hbm, m_t)
    pltpu.sync_copy(lvt_hbm, p_t)
    pltpu.sync_copy(lw_hbm, lw_v)

    nkc = _K // 16

    def prep(ll, carry):
        l = l0 + ll
        for kc in range(nkc):
            sl = pl.ds(16 * kc, 16)
            lw = lw_v[sl]
            lv = p_t[l, sl]
            a_t[l, sl] = (_C + lw) - 0.5 * lv
            p_t[l, sl] = 0.5 * jnp.exp(-lv)
        return carry

    lax.fori_loop(0, _LH, prep, 0)

    nj = _BPT // 16

    def lbody(ll, carry):
        l = l0 + ll
        zv = tuple(z_v[l, pl.ds(16 * j, 16)] for j in range(nj))

        def p1(kc, mxs):
            sl = pl.ds(16 * kc, 16)
            mv = m_t[l, sl]
            av = a_t[l, sl]
            pv = p_t[l, sl]
            mxs = list(mxs)
            for ic in range(4):
                for j in range(nj):
                    ts = []
                    for i in range(4 * ic, 4 * ic + 4):
                        m, a, p = mv[i], av[i], pv[i]
                        d = zv[j] - m
                        t = a - p * d * d
                        t_v[16 * kc + i, pl.ds(16 * j, 16)] = t
                        ts.append(t)
                    t01 = jnp.maximum(ts[0], ts[1])
                    t23 = jnp.maximum(ts[2], ts[3])
                    mxs[j] = jnp.maximum(mxs[j], jnp.maximum(t01, t23))
            return tuple(mxs)

        mxs = lax.fori_loop(
            0, nkc, p1,
            tuple(jnp.full((16,), _NEG, jnp.float32) for _ in range(nj)))

        def p2(kc, ss):
            ss = list(ss)
            for ic in range(4):
                for j in range(nj):
                    es = []
                    for i in range(4 * ic, 4 * ic + 4):
                        t = t_v[16 * kc + i, pl.ds(16 * j, 16)]
                        es.append(jnp.exp(t - mxs[j]))
                    ss[j] = ss[j] + ((es[0] + es[1]) + (es[2] + es[3]))
            return tuple(ss)

        ss = lax.fori_loop(
            0, nkc, p2, tuple(jnp.zeros((16,), jnp.float32) for _ in range(nj)),
            unroll=2)

        for j in range(nj):
            mx_v[ll, pl.ds(16 * j, 16)] = mxs[j]
            s_v[ll, pl.ds(16 * j, 16)] = ss[j]
        return carry

    lax.fori_loop(0, _LH, lbody, 0)
    pltpu.sync_copy(mx_v, mx_hbm.at[wid])
    pltpu.sync_copy(s_v, s_hbm.at[wid])


_sc_mog = functools.partial(
    pl.kernel,
    mesh=plsc.VectorSubcoreMesh(core_axis_name="c", subcore_axis_name="s"),
    out_type=[
        jax.ShapeDtypeStruct((_NTILES, _LH, _BPT), jnp.float32),
        jax.ShapeDtypeStruct((_NTILES, _LH, _BPT), jnp.float32),
    ],
    scratch_types=[
        pltpu.VMEM((_L, _BPT), jnp.float32),
        pltpu.VMEM((_L, _K), jnp.float32),
        pltpu.VMEM((_L, _K), jnp.float32),
        pltpu.VMEM((_L, _K), jnp.float32),
        pltpu.VMEM((_K,), jnp.float32),
        pltpu.VMEM((_LH, _BPT), jnp.float32),
        pltpu.VMEM((_LH, _BPT), jnp.float32),
        pltpu.VMEM((_K, _BPT), jnp.float32),
    ],
)(_sc_body)


_NB = _TC_ROWS
_TC_GRID = _TC_ROWS // _NB


def _tc_body(zt_ref, mt_ref, lvt_ref, lw_ref, o_ref, A_ref, B_ref, C_ref):
    mt = mt_ref[...]
    lvt = lvt_ref[...]
    lw = lw_ref[...]
    wmax = jnp.max(lw)
    lse_w = wmax + jnp.log(jnp.sum(jnp.exp(lw - wmax)))
    p = 0.5 * jnp.exp(-lvt)
    a = (_C + lw) - 0.5 * lvt
    A_ref[...] = a - p * mt * mt
    B_ref[...] = (2.0 * p) * mt
    C_ref[...] = -p

    def lstep(l, carry):
        zrow = zt_ref[pl.ds(l, 1), pl.ds(_SC_ROWS, _NB)]
        zsq = zrow * zrow
        ones = jnp.ones_like(zrow)
        zf = jnp.concatenate([ones, zrow, zsq], axis=0)
        wl = jnp.concatenate([A_ref[pl.ds(l, 1), :],
                              B_ref[pl.ds(l, 1), :],
                              C_ref[pl.ds(l, 1), :]], axis=0)
        t = lax.dot_general(wl, zf, (((0,), (0,)), ((), ())),
                            preferred_element_type=jnp.float32)
        mx = jnp.max(t, axis=0, keepdims=True)
        s = jnp.sum(jnp.exp(t - mx), axis=0, keepdims=True)
        o_ref[pl.ds(l, 1), :] = (mx - lse_w) + jnp.log(s)
        return carry

    lax.fori_loop(0, _L, lstep, 0, unroll=2)


def _tc_main(zt, mt, lvt, lwr):
    return pl.pallas_call(
        _tc_body,
        grid=(_TC_GRID,),
        in_specs=[
            pl.BlockSpec((_L, _B), lambda i: (0, 0)),
            pl.BlockSpec((_L, _K), lambda i: (0, 0)),
            pl.BlockSpec((_L, _K), lambda i: (0, 0)),
            pl.BlockSpec((1, _K), lambda i: (0, 0)),
        ],
        out_specs=pl.BlockSpec((_L, _NB), lambda i: (0, i)),
        out_shape=jax.ShapeDtypeStruct((_L, _TC_ROWS), jnp.float32),
        scratch_shapes=[
            pltpu.VMEM((_L, _K), jnp.float32),
            pltpu.VMEM((_L, _K), jnp.float32),
            pltpu.VMEM((_L, _K), jnp.float32),
        ],
    )(zt, mt, lvt, lwr)


def _fin_body(mx_ref, s_ref, w_ref, o_ref):
    lw = w_ref[...]
    wmax = jnp.max(lw)
    lse_w = wmax + jnp.log(jnp.sum(jnp.exp(lw - wmax)))
    o_ref[...] = (mx_ref[...] - lse_w) + jnp.log(s_ref[...])


def _finish(mx2, s2, wr):
    return pl.pallas_call(
        _fin_body,
        out_shape=jax.ShapeDtypeStruct(mx2.shape, jnp.float32),
    )(mx2, s2, wr)


@jax.jit
def kernel(z, means, logvars, w):
    ws = w.reshape(_K)

    zt = z.T
    mt = means.T
    lvt = logvars.T

    zt3 = z[:_SC_ROWS].reshape(_NBG, _BPT, _L).transpose(0, 2, 1)
    mx3, s3 = _sc_mog(zt3, mt, lvt, ws)

    out_tc = _tc_main(zt, mt, lvt, ws.reshape(1, _K)).T

    out_sc = (_finish(mx3.reshape(-1, _LANES), s3.reshape(-1, _LANES),
                      ws.reshape(1, _K))
              .reshape(_NBG, 2, _LH, _BPT)
              .transpose(0, 3, 1, 2)
              .reshape(_SC_ROWS, _L))
    return jnp.concatenate([out_sc, out_tc], axis=0)
